# baseline (speedup 1.0000x reference)
import jax
import jax.numpy as jnp
from jax.experimental import pallas as pl
from jax.experimental.pallas import tpu as pltpu

D_MIN = 0.001
LAMB = 0.85
ALPHA = 10.0
BETA1 = 0.1
BETA2 = 0.1

_ROWS = 600
_N = 256
_B = 2


def _loss_kernel(o_ref, d_ref, l_ref, c_ref, out_ref):
    o = o_ref[...]
    dd = d_ref[...]
    m = jnp.logical_and(o >= D_MIN, dd >= D_MIN)
    mf = m.astype(jnp.float32)
    g = jnp.log(o * mf + 0.001) - jnp.log(dd * mf + 0.001)
    s1 = jnp.sum(g)
    s2 = jnp.sum(g * g)
    n_el = jnp.float32(_B * _ROWS * 128)
    mean = s1 / n_el
    var = (s2 - n_el * mean * mean) / (n_el - 1.0)
    sil = jnp.sqrt(var + (1.0 - LAMB) * mean * mean)

    def chamfer_one(t_full, b):
        t = t_full[b * _ROWS:(b + 1) * _ROWS, :]
        valid = t >= D_MIN
        vf = valid.astype(jnp.float32)
        tsub = jnp.where(valid, t, 1e6)

        def body(c, carry):
            miny, sumx = carry
            cv = c_ref[b, c]
            d2 = (tsub - cv) ** 2
            miny = jnp.minimum(miny, d2)
            sumx = sumx + jnp.min(d2)
            return miny, sumx

        miny0 = jnp.full((_ROWS, 128), 1e30, jnp.float32)
        miny, sumx = jax.lax.fori_loop(0, _N, body, (miny0, jnp.float32(0.0)))
        sumy = jnp.sum(jnp.where(valid, miny, 0.0))
        cnt = jnp.sum(vf)
        cham_y = sumy / jnp.maximum(cnt, 1.0)
        return sumx / jnp.float32(_N), cham_y

    bc = []
    for t_full in (d_ref[...], l_ref[...]):
        cx0, cy0 = chamfer_one(t_full, 0)
        cx1, cy1 = chamfer_one(t_full, 1)
        bc.append(0.5 * (cx0 + cx1) + 0.5 * (cy0 + cy1))

    out_ref[0, 0] = ALPHA * sil + BETA1 * bc[0] + BETA2 * bc[1]


def kernel(output, centers, depth, lidar):
    o = output.reshape(_B * _ROWS, 128)
    d = depth.reshape(_B * _ROWS, 128)
    l = lidar.reshape(_B * _ROWS, 128)
    res = pl.pallas_call(
        _loss_kernel,
        out_shape=jax.ShapeDtypeStruct((1, 1), jnp.float32),
        in_specs=[
            pl.BlockSpec(memory_space=pltpu.VMEM),
            pl.BlockSpec(memory_space=pltpu.VMEM),
            pl.BlockSpec(memory_space=pltpu.VMEM),
            pl.BlockSpec(memory_space=pltpu.SMEM),
        ],
        out_specs=pl.BlockSpec(memory_space=pltpu.SMEM),
    )(o, d, l, centers)
    return res[0, 0]


# speedup vs baseline: 2.4936x; 2.4936x over previous
import functools

import jax
import jax.numpy as jnp
from jax import lax
from jax.experimental import pallas as pl
from jax.experimental.pallas import tpu as pltpu
from jax.experimental.pallas import tpu_sc as plsc

D_MIN = 0.001
LAMB = 0.85
ALPHA = 10.0
BETA1 = 0.1
BETA2 = 0.1

_B = 2
_N = 256
_P = 76800
_ROWS = 600
_NW = 32
_PPW = _P * 2 * _B // _NW
_NBIN = 272
_U = 2


def _sort_body(c_ref, ct_ref, out_ref):
    for b in range(_B):
        row = c_ref[b:b + 1, :]
        col = ct_ref[:, b:b + 1]
        a = jnp.broadcast_to(row, (_N, _N))
        bb = jnp.broadcast_to(col, (_N, _N))
        ii = lax.broadcasted_iota(jnp.int32, (_N, _N), 0)
        jj = lax.broadcasted_iota(jnp.int32, (_N, _N), 1)
        less = jnp.logical_or(a < bb, jnp.logical_and(a == bb, jj < ii))
        rank = jnp.sum(less.astype(jnp.int32), axis=1, keepdims=True)
        onehot = (rank == jj).astype(jnp.float32)
        out_ref[b:b + 1, :] = jnp.sum(col * onehot, axis=0, keepdims=True)


def _sort_centers(centers):
    return pl.pallas_call(
        _sort_body,
        out_shape=jax.ShapeDtypeStruct((_B, _N), jnp.float32),
        in_specs=[
            pl.BlockSpec(memory_space=pltpu.VMEM),
            pl.BlockSpec(memory_space=pltpu.VMEM),
        ],
        out_specs=pl.BlockSpec(memory_space=pltpu.VMEM),
    )(centers, centers.T)


def _sc_body(s_hbm, pts_hbm, mx_hbm, mn_hbm, sy_hbm, cy_hbm,
             cent_v, pts_v, mx_v, mn_v, sy_v, cy_v):
    cid = lax.axis_index("c")
    sid = lax.axis_index("s")
    wid = sid * 2 + cid
    combo = wid // 8
    sub = wid % 8
    b = combo % 2

    pltpu.sync_copy(s_hbm.at[b], cent_v)
    pltpu.sync_copy(pts_hbm.at[combo, pl.ds(sub * _PPW, _PPW)], pts_v)

    neg = jnp.full((16,), -1e6, jnp.float32)
    pos = jnp.full((16,), 1e6, jnp.float32)
    for k in range(_NBIN):
        mx_v[pl.ds(k * 16, 16)] = neg
        mn_v[pl.ds(k * 16, 16)] = pos

    iota = lax.iota(jnp.int32, 16)
    lane_base = iota * _NBIN
    smax = plsc.load_gather(cent_v, [jnp.full((16,), _N - 1, jnp.int32)])

    def body(g, carry):
        sumy, cnty = carry
        for u in range(_U):
            off = pl.multiple_of((g * _U + u) * 16, 16)
            t = pts_v[pl.ds(off, 16)]
            lo = jnp.zeros((16,), jnp.int32)
            for step in (128, 64, 32, 16, 8, 4, 2, 1):
                v = plsc.load_gather(cent_v, [lo + (step - 1)])
                lo = jnp.where(v <= t, lo + step, lo)
            lo = lo + jnp.where(smax <= t, 1, 0)
            pv = plsc.load_gather(cent_v, [jnp.maximum(lo - 1, 0)])
            sv = plsc.load_gather(cent_v, [jnp.minimum(lo, _N - 1)])
            ep = t - pv
            es = sv - t
            dpred = jnp.where(lo > 0, ep * ep, 1e30)
            dsucc = jnp.where(lo < _N, es * es, 1e30)
            dy = jnp.minimum(dpred, dsucc)
            valid = t >= D_MIN
            sumy = sumy + jnp.where(valid, dy, 0.0)
            cnty = cnty + jnp.where(valid, 1.0, 0.0)
            idx = lane_base + lo
            vx = jnp.where(valid, t, -1e6)
            vn = jnp.where(valid, t, 1e6)
            oldx = plsc.load_gather(mx_v, [idx])
            plsc.store_scatter(mx_v, [idx], jnp.maximum(oldx, vx))
            oldn = plsc.load_gather(mn_v, [idx])
            plsc.store_scatter(mn_v, [idx], jnp.minimum(oldn, vn))
        return sumy, cnty

    z = jnp.zeros((16,), jnp.float32)
    sumy, cnty = lax.fori_loop(0, _PPW // 16 // _U, body, (z, z))
    sy_v[...] = sumy
    cy_v[...] = cnty
    pltpu.sync_copy(mx_v, mx_hbm.at[wid])
    pltpu.sync_copy(mn_v, mn_hbm.at[wid])
    pltpu.sync_copy(sy_v, sy_hbm.at[wid])
    pltpu.sync_copy(cy_v, cy_hbm.at[wid])


def _sc_chamfer(sorted_centers, pts):
    mesh = plsc.VectorSubcoreMesh(core_axis_name="c", subcore_axis_name="s")
    f = pl.kernel(
        _sc_body,
        out_type=[
            jax.ShapeDtypeStruct((_NW, 16 * _NBIN), jnp.float32),
            jax.ShapeDtypeStruct((_NW, 16 * _NBIN), jnp.float32),
            jax.ShapeDtypeStruct((_NW, 16), jnp.float32),
            jax.ShapeDtypeStruct((_NW, 16), jnp.float32),
        ],
        mesh=mesh,
        compiler_params=pltpu.CompilerParams(needs_layout_passes=False),
        scratch_types=[
            pltpu.VMEM((_N,), jnp.float32),
            pltpu.VMEM((_PPW,), jnp.float32),
            pltpu.VMEM((16 * _NBIN,), jnp.float32),
            pltpu.VMEM((16 * _NBIN,), jnp.float32),
            pltpu.VMEM((16,), jnp.float32),
            pltpu.VMEM((16,), jnp.float32),
        ],
    )
    return f(sorted_centers, pts)


def _finish_body(o_ref, d_ref, st_ref, mx_ref, mn_ref, sy_ref, cy_ref, out_ref):
    o = o_ref[...]
    dd = d_ref[...]
    m = jnp.logical_and(o >= D_MIN, dd >= D_MIN).astype(jnp.float32)
    g = jnp.log(o * m + 0.001) - jnp.log(dd * m + 0.001)
    s1 = jnp.sum(g)
    s2 = jnp.sum(g * g)
    n_el = jnp.float32(_B * _P)
    mean = s1 / n_el
    var = (s2 - n_el * mean * mean) / (n_el - 1.0)
    sil = jnp.sqrt(var + (1.0 - LAMB) * mean * mean)

    kk = lax.broadcasted_iota(jnp.int32, (_N, _NBIN), 0)
    jj = lax.broadcasted_iota(jnp.int32, (_N, _NBIN), 1)
    pmask = jj <= kk
    smask = jj >= kk + 1

    sumx = []
    chamy = []
    for c in range(4):
        bmx = jnp.max(mx_ref[c], axis=0, keepdims=True)
        bmn = jnp.min(mn_ref[c], axis=0, keepdims=True)
        pred = jnp.max(jnp.where(pmask, jnp.broadcast_to(bmx, (_N, _NBIN)), -1e9),
                       axis=1, keepdims=True)
        succ = jnp.min(jnp.where(smask, jnp.broadcast_to(bmn, (_N, _NBIN)), 1e9),
                       axis=1, keepdims=True)
        sb = st_ref[:, (c % 2):(c % 2) + 1]
        minx = jnp.minimum((sb - pred) ** 2, (succ - sb) ** 2)
        sumx.append(jnp.sum(minx))
        sy = jnp.sum(sy_ref[c * 8:(c + 1) * 8, :])
        cy = jnp.sum(cy_ref[c * 8:(c + 1) * 8, :])
        chamy.append(sy / jnp.maximum(cy, 1.0))

    bc_d = 0.5 * (sumx[0] + sumx[1]) / _N + 0.5 * (chamy[0] + chamy[1])
    bc_l = 0.5 * (sumx[2] + sumx[3]) / _N + 0.5 * (chamy[2] + chamy[3])
    out_ref[0, 0] = ALPHA * sil + BETA1 * bc_d + BETA2 * bc_l


def _finish(o, d, st, mx, mn, sy, cy):
    return pl.pallas_call(
        _finish_body,
        out_shape=jax.ShapeDtypeStruct((1, 1), jnp.float32),
        in_specs=[pl.BlockSpec(memory_space=pltpu.VMEM)] * 7,
        out_specs=pl.BlockSpec(memory_space=pltpu.SMEM),
    )(o, d, st, mx, mn, sy, cy)


def kernel(output, centers, depth, lidar):
    o = output.reshape(_B * _ROWS, 128)
    d = depth.reshape(_B * _ROWS, 128)
    pts = jnp.concatenate(
        [depth.reshape(_B, _P), lidar.reshape(_B, _P)], axis=0)
    s = _sort_centers(centers)
    mx, mn, sy, cy = _sc_chamfer(s, pts)
    mx4 = mx.reshape(4, 8 * 16, _NBIN)
    mn4 = mn.reshape(4, 8 * 16, _NBIN)
    res = _finish(o, d, s.T, mx4, mn4, sy, cy)
    return res[0, 0]
